# Initial kernel scaffold; baseline (speedup 1.0000x reference)
import jax
import jax.numpy as jnp
from jax.experimental import pallas as pl


def kernel(node_feats, segment_ids, W_embed, b_embed, W_out, b_out):
    raise NotImplementedError("write your pallas kernel here")


# speedup vs baseline: 5.6891x; 5.6891x over previous
import functools

import jax
import jax.numpy as jnp
from jax import lax
from jax.experimental import pallas as pl
from jax.experimental.pallas import tpu as pltpu
from jax.experimental.pallas import tpu_sc as plsc

N = 100000
D = 128
NUM_SEGMENTS = 512
NUM_PIECES = 5

NC = 2
NS = 16
NW = NC * NS
L = 16
TILE = 128
NTILES = N // TILE
TAIL = N - NTILES * TILE
ROWS = NUM_SEGMENTS // NS

_mesh = plsc.VectorSubcoreMesh(
    core_axis_name="c", subcore_axis_name="s", num_cores=NC, num_subcores=NS
)


@functools.partial(
    pl.kernel,
    out_type=(
        jax.ShapeDtypeStruct((NC, NUM_SEGMENTS, D), jnp.float32),
        jax.ShapeDtypeStruct((NW, NUM_SEGMENTS), jnp.float32),
    ),
    mesh=_mesh,
    scratch_types=(
        pltpu.VMEM((TILE, D), jnp.float32),
        pltpu.VMEM((TILE,), jnp.int32),
        pltpu.VMEM((TAIL,), jnp.int32),
        pltpu.VMEM((NUM_SEGMENTS,), jnp.float32),
        pltpu.VMEM_SHARED((NUM_SEGMENTS, D), jnp.float32),
    ),
    compiler_params=pltpu.CompilerParams(needs_layout_passes=False),
)
def _segment_sums(feats, ids, zeros, out_sums, out_counts,
                  buf, idbuf, tail_ids, hist, acc_sh):
    cid = lax.axis_index("c")
    sid = lax.axis_index("s")
    wid = cid * NS + sid

    pltpu.sync_copy(zeros.at[pl.ds(sid * ROWS, ROWS), :], buf.at[pl.ds(0, ROWS), :])
    pltpu.sync_copy(buf.at[pl.ds(0, ROWS), :], acc_sh.at[pl.ds(sid * ROWS, ROWS), :])
    for k in range(NUM_SEGMENTS // L):
        hist[pl.ds(k * L, L)] = jnp.zeros((L,), jnp.float32)
    plsc.subcore_barrier()

    ones = jnp.full((L,), 1.0, jnp.float32)

    nt = (NTILES - wid + NW - 1) // NW

    def body(i, carry):
        r0 = (wid + i * NW) * TILE
        pltpu.sync_copy(feats.at[pl.ds(r0, TILE), :], buf)
        pltpu.sync_copy(ids.at[pl.ds(r0, TILE)], idbuf)
        pltpu.sync_copy(buf, acc_sh.at[idbuf], add=True)
        for k in range(TILE // L):
            plsc.addupdate_scatter(hist, [idbuf[pl.ds(k * L, L)]], ones)
        return carry

    lax.fori_loop(0, nt, body, 0)

    @pl.when(wid == NW - 1)
    def _tail():
        r0 = NTILES * TILE
        pltpu.sync_copy(feats.at[pl.ds(r0, TAIL), :], buf.at[pl.ds(0, TAIL), :])
        pltpu.sync_copy(ids.at[pl.ds(r0, TAIL)], tail_ids)
        pltpu.sync_copy(buf.at[pl.ds(0, TAIL), :], acc_sh.at[tail_ids], add=True)
        for k in range(TAIL // L):
            plsc.addupdate_scatter(hist, [tail_ids[pl.ds(k * L, L)]], ones)

    plsc.subcore_barrier()

    pltpu.sync_copy(acc_sh.at[pl.ds(sid * ROWS, ROWS), :], buf.at[pl.ds(0, ROWS), :])
    pltpu.sync_copy(buf.at[pl.ds(0, ROWS), :],
                    out_sums.at[cid, pl.ds(sid * ROWS, ROWS), :])
    pltpu.sync_copy(hist, out_counts.at[wid])


def _finish_body(s0, s1, cnt, we, be, wo, bo, out):
    sums = s0[...] + s1[...]
    counts = lax.dot_general(cnt[...], jnp.ones((NW, 1), jnp.float32),
                             (((0,), (0,)), ((), ())),
                             precision=lax.Precision.HIGHEST,
                             preferred_element_type=jnp.float32)
    mean = sums * (1.0 / jnp.maximum(counts, 1.0))
    g = lax.dot_general(mean, we[...], (((1,), (0,)), ((), ())),
                        precision=lax.Precision.HIGHEST,
                        preferred_element_type=jnp.float32)
    g = g + jnp.where(counts > 0.0, 1.0, 0.0) * be[...]
    out[...] = lax.dot_general(g, wo[...], (((1,), (0,)), ((), ())),
                               precision=lax.Precision.HIGHEST,
                               preferred_element_type=jnp.float32) + bo[...]


_finish = pl.pallas_call(
    _finish_body,
    out_shape=jax.ShapeDtypeStruct((NUM_SEGMENTS, NUM_PIECES), jnp.float32),
)


def kernel(node_feats, segment_ids, W_embed, b_embed, W_out, b_out):
    zeros = jnp.zeros((NUM_SEGMENTS, D), jnp.float32)
    sums, counts = _segment_sums(node_feats, segment_ids, zeros)
    return _finish(
        sums[0], sums[1], counts,
        W_embed, b_embed.reshape(1, D),
        W_out, b_out.reshape(1, NUM_PIECES),
    )


# speedup vs baseline: 8.3493x; 1.4676x over previous
import functools

import jax
import jax.numpy as jnp
from jax import lax
from jax.experimental import pallas as pl
from jax.experimental.pallas import tpu as pltpu
from jax.experimental.pallas import tpu_sc as plsc

N = 100000
D = 128
NUM_SEGMENTS = 512
NUM_PIECES = 5

NC = 2
NS = 16
NW = NC * NS
L = 16
TILE = 128
NTILES = N // TILE
TAIL = N - NTILES * TILE
ROWS = NUM_SEGMENTS // NS
BASE = NTILES // NW
EXTRA = NTILES - BASE * NW
Q = 32
IDROWS = 784

_mesh = plsc.VectorSubcoreMesh(
    core_axis_name="c", subcore_axis_name="s", num_cores=NC, num_subcores=NS
)


@functools.partial(
    pl.kernel,
    out_type=(
        jax.ShapeDtypeStruct((NC, NUM_SEGMENTS, D), jnp.float32),
        jax.ShapeDtypeStruct((NW, NUM_SEGMENTS), jnp.float32),
    ),
    mesh=_mesh,
    scratch_types=(
        pltpu.VMEM((TILE, D), jnp.float32),
        pltpu.VMEM((TILE, D), jnp.float32),
        pltpu.VMEM((Q, TILE), jnp.int32),
        pltpu.VMEM((TAIL,), jnp.int32),
        pltpu.VMEM((NUM_SEGMENTS,), jnp.float32),
        pltpu.VMEM_SHARED((NUM_SEGMENTS, D), jnp.float32),
        pltpu.SemaphoreType.DMA,
        pltpu.SemaphoreType.DMA,
    ),
    compiler_params=pltpu.CompilerParams(needs_layout_passes=False),
)
def _segment_sums(feats, ids, ids2, zeros, out_sums, out_counts,
                  buf0, buf1, idsw, tail_ids, hist, acc_sh, sem0, sem1):
    cid = lax.axis_index("c")
    sid = lax.axis_index("s")
    wid = cid * NS + sid

    pltpu.sync_copy(zeros.at[pl.ds(sid * ROWS, ROWS), :], buf0.at[pl.ds(0, ROWS), :])
    pltpu.sync_copy(buf0.at[pl.ds(0, ROWS), :], acc_sh.at[pl.ds(sid * ROWS, ROWS), :])
    for k in range(NUM_SEGMENTS // L):
        hist[pl.ds(k * L, L)] = jnp.zeros((L,), jnp.float32)
    plsc.subcore_barrier()

    ones = jnp.full((L,), 1.0, jnp.float32)

    start = wid * BASE + jnp.minimum(wid, EXTRA)
    nt = BASE + jnp.where(wid < EXTRA, 1, 0)

    abase = (start // 8) * 8
    off = start - abase
    pltpu.sync_copy(ids2.at[pl.ds(abase, Q), :], idsw)

    def load(i, b, sem):
        pltpu.async_copy(feats.at[pl.ds((start + i) * TILE, TILE), :], b, sem)

    def wait(i, b, sem):
        pltpu.make_async_copy(
            feats.at[pl.ds((start + i) * TILE, TILE), :], b, sem).wait()

    def scatter(i, b):
        pltpu.sync_copy(b, acc_sh.at[idsw.at[off + i]], add=True)

    def count(i):
        for k in range(TILE // L):
            plsc.addupdate_scatter(hist, [idsw[off + i, pl.ds(k * L, L)]], ones)

    load(0, buf0, sem0)
    load(1, buf1, sem1)

    def pair(p, carry):
        i0 = 2 * p
        wait(i0, buf0, sem0)
        scatter(i0, buf0)

        @pl.when(i0 + 2 < nt)
        def _():
            load(i0 + 2, buf0, sem0)

        count(i0)

        i1 = 2 * p + 1
        wait(i1, buf1, sem1)
        scatter(i1, buf1)

        @pl.when(i1 + 2 < nt)
        def _():
            load(i1 + 2, buf1, sem1)

        count(i1)
        return carry

    lax.fori_loop(0, BASE // 2, pair, 0)

    @pl.when(nt > BASE)
    def _last():
        wait(BASE, buf0, sem0)
        scatter(BASE, buf0)
        count(BASE)

    @pl.when(wid == NW - 1)
    def _tail():
        r0 = NTILES * TILE
        pltpu.sync_copy(feats.at[pl.ds(r0, TAIL), :], buf0.at[pl.ds(0, TAIL), :])
        pltpu.sync_copy(ids.at[pl.ds(r0, TAIL)], tail_ids)
        pltpu.sync_copy(buf0.at[pl.ds(0, TAIL), :], acc_sh.at[tail_ids], add=True)
        for k in range(TAIL // L):
            plsc.addupdate_scatter(hist, [tail_ids[pl.ds(k * L, L)]], ones)

    plsc.subcore_barrier()

    pltpu.sync_copy(acc_sh.at[pl.ds(sid * ROWS, ROWS), :], buf0.at[pl.ds(0, ROWS), :])
    pltpu.sync_copy(buf0.at[pl.ds(0, ROWS), :],
                    out_sums.at[cid, pl.ds(sid * ROWS, ROWS), :])
    pltpu.sync_copy(hist, out_counts.at[wid])


def _finish_body(s0, s1, cnt, we, be, wo, bo, out):
    sums = s0[...] + s1[...]
    counts = lax.dot_general(cnt[...], jnp.ones((NW, 1), jnp.float32),
                             (((0,), (0,)), ((), ())),
                             precision=lax.Precision.HIGHEST,
                             preferred_element_type=jnp.float32)
    mean = sums * (1.0 / jnp.maximum(counts, 1.0))
    g = lax.dot_general(mean, we[...], (((1,), (0,)), ((), ())),
                        precision=lax.Precision.HIGHEST,
                        preferred_element_type=jnp.float32)
    g = g + jnp.where(counts > 0.0, 1.0, 0.0) * be[...]
    out[...] = lax.dot_general(g, wo[...], (((1,), (0,)), ((), ())),
                               precision=lax.Precision.HIGHEST,
                               preferred_element_type=jnp.float32) + bo[...]


_finish = pl.pallas_call(
    _finish_body,
    out_shape=jax.ShapeDtypeStruct((NUM_SEGMENTS, NUM_PIECES), jnp.float32),
)


def kernel(node_feats, segment_ids, W_embed, b_embed, W_out, b_out):
    zeros = jnp.zeros((NUM_SEGMENTS, D), jnp.float32)
    ids2 = jnp.pad(segment_ids, (0, IDROWS * TILE - N)).reshape(IDROWS, TILE)
    sums, counts = _segment_sums(node_feats, segment_ids, ids2, zeros)
    return _finish(
        sums[0], sums[1], counts,
        W_embed, b_embed.reshape(1, D),
        W_out, b_out.reshape(1, NUM_PIECES),
    )


# speedup vs baseline: 8.5538x; 1.0245x over previous
import functools

import jax
import jax.numpy as jnp
from jax import lax
from jax.experimental import pallas as pl
from jax.experimental.pallas import tpu as pltpu
from jax.experimental.pallas import tpu_sc as plsc

N = 100000
D = 128
NUM_SEGMENTS = 512
NUM_PIECES = 5

NC = 2
NS = 16
NW = NC * NS
L = 16
TILE = 128
NTILES = N // TILE
TAIL = N - NTILES * TILE
ROWS = NUM_SEGMENTS // NS
BASE = NTILES // NW
EXTRA = NTILES - BASE * NW
Q = 32
IDROWS = 784

_mesh = plsc.VectorSubcoreMesh(
    core_axis_name="c", subcore_axis_name="s", num_cores=NC, num_subcores=NS
)


@functools.partial(
    pl.kernel,
    out_type=(
        jax.ShapeDtypeStruct((NC, NUM_SEGMENTS, D), jnp.float32),
        jax.ShapeDtypeStruct((NW, NUM_SEGMENTS), jnp.float32),
    ),
    mesh=_mesh,
    scratch_types=(
        pltpu.VMEM((TILE, D), jnp.float32),
        pltpu.VMEM((TILE, D), jnp.float32),
        pltpu.VMEM((TILE, D), jnp.float32),
        pltpu.VMEM((TILE, D), jnp.float32),
        pltpu.VMEM((Q, TILE), jnp.int32),
        pltpu.VMEM((TAIL,), jnp.int32),
        pltpu.VMEM((NUM_SEGMENTS,), jnp.float32),
        pltpu.VMEM_SHARED((NUM_SEGMENTS, D), jnp.float32),
        pltpu.SemaphoreType.DMA,
        pltpu.SemaphoreType.DMA,
        pltpu.SemaphoreType.DMA,
        pltpu.SemaphoreType.DMA,
        pltpu.SemaphoreType.DMA,
        pltpu.SemaphoreType.DMA,
        pltpu.SemaphoreType.DMA,
        pltpu.SemaphoreType.DMA,
    ),
    compiler_params=pltpu.CompilerParams(needs_layout_passes=False),
)
def _segment_sums(feats, ids, ids2, zeros, out_sums, out_counts,
                  buf0, buf1, buf2, buf3, idsw, tail_ids, hist, acc_sh,
                  ls0, ls1, ls2, ls3, ss0, ss1, ss2, ss3):
    bufs = (buf0, buf1, buf2, buf3)
    lsems = (ls0, ls1, ls2, ls3)
    ssems = (ss0, ss1, ss2, ss3)
    cid = lax.axis_index("c")
    sid = lax.axis_index("s")
    wid = cid * NS + sid

    pltpu.sync_copy(zeros.at[pl.ds(sid * ROWS, ROWS), :], buf0.at[pl.ds(0, ROWS), :])
    pltpu.sync_copy(buf0.at[pl.ds(0, ROWS), :], acc_sh.at[pl.ds(sid * ROWS, ROWS), :])
    for k in range(NUM_SEGMENTS // L):
        hist[pl.ds(k * L, L)] = jnp.zeros((L,), jnp.float32)
    plsc.subcore_barrier()

    ones = jnp.full((L,), 1.0, jnp.float32)

    start = wid * BASE + jnp.minimum(wid, EXTRA)
    nt = BASE + jnp.where(wid < EXTRA, 1, 0)

    abase = (start // 8) * 8
    off = start - abase
    pltpu.sync_copy(ids2.at[pl.ds(abase, Q), :], idsw)

    def load(i, j):
        pltpu.async_copy(feats.at[pl.ds((start + i) * TILE, TILE), :],
                         bufs[j], lsems[j])

    def wait_load(i, j):
        pltpu.make_async_copy(feats.at[pl.ds((start + i) * TILE, TILE), :],
                              bufs[j], lsems[j]).wait()

    def scatter(i, j):
        pltpu.async_copy(bufs[j], acc_sh.at[idsw.at[off + i]], ssems[j],
                         add=True)

    def wait_scatter(i, j):
        pltpu.make_async_copy(bufs[j], acc_sh.at[idsw.at[off + i]],
                              ssems[j]).wait()

    def count(i):
        for k in range(TILE // L):
            plsc.addupdate_scatter(hist, [idsw[off + i, pl.ds(k * L, L)]], ones)

    def step(i, j):
        wait_load(i, j)
        scatter(i, j)
        count(i)
        nj = (j + 2) % 4

        @pl.when(i >= 2)
        def _():
            wait_scatter(i - 2, nj)

        @pl.when(i + 2 < nt)
        def _():
            load(i + 2, nj)

    load(0, 0)
    load(1, 1)

    def quad(q, carry):
        for j in range(4):
            step(4 * q + j, j)
        return carry

    lax.fori_loop(0, BASE // 4, quad, 0)

    @pl.when(nt > BASE)
    def _last():
        step(BASE, BASE % 4)

    @pl.when(nt == BASE)
    def _drain_even():
        wait_scatter(BASE - 2, (BASE - 2) % 4)
        wait_scatter(BASE - 1, (BASE - 1) % 4)

    @pl.when(nt > BASE)
    def _drain_odd():
        wait_scatter(BASE - 1, (BASE - 1) % 4)
        wait_scatter(BASE, BASE % 4)

    @pl.when(wid == NW - 1)
    def _tail():
        r0 = NTILES * TILE
        pltpu.sync_copy(feats.at[pl.ds(r0, TAIL), :], buf0.at[pl.ds(0, TAIL), :])
        pltpu.sync_copy(ids.at[pl.ds(r0, TAIL)], tail_ids)
        pltpu.sync_copy(buf0.at[pl.ds(0, TAIL), :], acc_sh.at[tail_ids], add=True)
        for k in range(TAIL // L):
            plsc.addupdate_scatter(hist, [tail_ids[pl.ds(k * L, L)]], ones)

    plsc.subcore_barrier()

    pltpu.sync_copy(acc_sh.at[pl.ds(sid * ROWS, ROWS), :], buf0.at[pl.ds(0, ROWS), :])
    pltpu.sync_copy(buf0.at[pl.ds(0, ROWS), :],
                    out_sums.at[cid, pl.ds(sid * ROWS, ROWS), :])
    pltpu.sync_copy(hist, out_counts.at[wid])


def _finish_body(s0, s1, cnt, we, be, wo, bo, out):
    sums = s0[...] + s1[...]
    counts = lax.dot_general(cnt[...], jnp.ones((NW, 1), jnp.float32),
                             (((0,), (0,)), ((), ())),
                             precision=lax.Precision.HIGHEST,
                             preferred_element_type=jnp.float32)
    mean = sums * (1.0 / jnp.maximum(counts, 1.0))
    g = lax.dot_general(mean, we[...], (((1,), (0,)), ((), ())),
                        precision=lax.Precision.HIGHEST,
                        preferred_element_type=jnp.float32)
    g = g + jnp.where(counts > 0.0, 1.0, 0.0) * be[...]
    out[...] = lax.dot_general(g, wo[...], (((1,), (0,)), ((), ())),
                               precision=lax.Precision.HIGHEST,
                               preferred_element_type=jnp.float32) + bo[...]


_finish = pl.pallas_call(
    _finish_body,
    out_shape=jax.ShapeDtypeStruct((NUM_SEGMENTS, NUM_PIECES), jnp.float32),
)


def kernel(node_feats, segment_ids, W_embed, b_embed, W_out, b_out):
    zeros = jnp.zeros((NUM_SEGMENTS, D), jnp.float32)
    ids2 = jnp.pad(segment_ids, (0, IDROWS * TILE - N)).reshape(IDROWS, TILE)
    sums, counts = _segment_sums(node_feats, segment_ids, ids2, zeros)
    return _finish(
        sums[0], sums[1], counts,
        W_embed, b_embed.reshape(1, D),
        W_out, b_out.reshape(1, NUM_PIECES),
    )


# speedup vs baseline: 8.9291x; 1.0439x over previous
import functools

import jax
import jax.numpy as jnp
from jax import lax
from jax.experimental import pallas as pl
from jax.experimental.pallas import tpu as pltpu
from jax.experimental.pallas import tpu_sc as plsc

N = 100000
D = 128
NUM_SEGMENTS = 512
NUM_PIECES = 5

NC = 2
NS = 16
NW = NC * NS
L = 16
TILE = 128
NTILES = N // TILE
TAIL = N - NTILES * TILE
ROWS = NUM_SEGMENTS // NS
BASE = NTILES // NW
EXTRA = NTILES - BASE * NW
Q = 32
IDROWS = 784

_mesh = plsc.VectorSubcoreMesh(
    core_axis_name="c", subcore_axis_name="s", num_cores=NC, num_subcores=NS
)


@functools.partial(
    pl.kernel,
    out_type=(
        jax.ShapeDtypeStruct((NC, NUM_SEGMENTS, D), jnp.float32),
        jax.ShapeDtypeStruct((NW, NUM_SEGMENTS), jnp.float32),
    ),
    mesh=_mesh,
    scratch_types=(
        pltpu.VMEM((TILE, D), jnp.float32),
        pltpu.VMEM((TILE, D), jnp.float32),
        pltpu.VMEM((TILE, D), jnp.float32),
        pltpu.VMEM((TILE, D), jnp.float32),
        pltpu.VMEM((Q, TILE), jnp.int32),
        pltpu.VMEM((TAIL,), jnp.int32),
        pltpu.VMEM((NUM_SEGMENTS,), jnp.float32),
        pltpu.VMEM_SHARED((NUM_SEGMENTS, D), jnp.float32),
        pltpu.SemaphoreType.DMA,
        pltpu.SemaphoreType.DMA,
        pltpu.SemaphoreType.DMA,
        pltpu.SemaphoreType.DMA,
        pltpu.SemaphoreType.DMA,
        pltpu.SemaphoreType.DMA,
        pltpu.SemaphoreType.DMA,
        pltpu.SemaphoreType.DMA,
    ),
    compiler_params=pltpu.CompilerParams(needs_layout_passes=False),
)
def _segment_sums(feats, ids, ids2, out_sums, out_counts,
                  buf0, buf1, buf2, buf3, idsw, tail_ids, hist, acc_sh,
                  ls0, ls1, ls2, ls3, ss0, ss1, ss2, ss3):
    bufs = (buf0, buf1, buf2, buf3)
    lsems = (ls0, ls1, ls2, ls3)
    ssems = (ss0, ss1, ss2, ss3)
    cid = lax.axis_index("c")
    sid = lax.axis_index("s")
    wid = cid * NS + sid

    z16 = jnp.zeros((L,), jnp.float32)

    def zrow(r, carry):
        for k in range(D // L):
            buf0[r, pl.ds(k * L, L)] = z16
        return carry

    lax.fori_loop(0, ROWS, zrow, 0)
    pltpu.sync_copy(buf0.at[pl.ds(0, ROWS), :], acc_sh.at[pl.ds(sid * ROWS, ROWS), :])
    for k in range(NUM_SEGMENTS // L):
        hist[pl.ds(k * L, L)] = z16
    plsc.subcore_barrier()

    ones = jnp.full((L,), 1.0, jnp.float32)

    start = wid * BASE + jnp.minimum(wid, EXTRA)
    nt = BASE + jnp.where(wid < EXTRA, 1, 0)

    abase = (start // 8) * 8
    off = start - abase
    pltpu.sync_copy(ids2.at[pl.ds(abase, Q), :], idsw)

    def load(i, j):
        pltpu.async_copy(feats.at[pl.ds((start + i) * TILE, TILE), :],
                         bufs[j], lsems[j])

    def wait_load(i, j):
        pltpu.make_async_copy(feats.at[pl.ds((start + i) * TILE, TILE), :],
                              bufs[j], lsems[j]).wait()

    def scatter(i, j):
        pltpu.async_copy(bufs[j], acc_sh.at[idsw.at[off + i]], ssems[j],
                         add=True)

    def wait_scatter(i, j):
        pltpu.make_async_copy(bufs[j], acc_sh.at[idsw.at[off + i]],
                              ssems[j]).wait()

    def count(i):
        for k in range(TILE // L):
            plsc.addupdate_scatter(hist, [idsw[off + i, pl.ds(k * L, L)]], ones)

    def step(i, j):
        wait_load(i, j)
        scatter(i, j)
        count(i)
        nj = (j + 2) % 4

        @pl.when(i >= 2)
        def _():
            wait_scatter(i - 2, nj)

        @pl.when(i + 2 < nt)
        def _():
            load(i + 2, nj)

    load(0, 0)
    load(1, 1)

    def quad(q, carry):
        for j in range(4):
            step(4 * q + j, j)
        return carry

    lax.fori_loop(0, BASE // 4, quad, 0)

    @pl.when(nt > BASE)
    def _last():
        step(BASE, BASE % 4)

    @pl.when(nt == BASE)
    def _drain_even():
        wait_scatter(BASE - 2, (BASE - 2) % 4)
        wait_scatter(BASE - 1, (BASE - 1) % 4)

    @pl.when(nt > BASE)
    def _drain_odd():
        wait_scatter(BASE - 1, (BASE - 1) % 4)
        wait_scatter(BASE, BASE % 4)

    @pl.when(wid == NW - 1)
    def _tail():
        r0 = NTILES * TILE
        pltpu.sync_copy(feats.at[pl.ds(r0, TAIL), :], buf0.at[pl.ds(0, TAIL), :])
        pltpu.sync_copy(ids.at[pl.ds(r0, TAIL)], tail_ids)
        pltpu.sync_copy(buf0.at[pl.ds(0, TAIL), :], acc_sh.at[tail_ids], add=True)
        for k in range(TAIL // L):
            plsc.addupdate_scatter(hist, [tail_ids[pl.ds(k * L, L)]], ones)

    plsc.subcore_barrier()

    pltpu.sync_copy(acc_sh.at[pl.ds(sid * ROWS, ROWS), :], buf0.at[pl.ds(0, ROWS), :])
    pltpu.sync_copy(buf0.at[pl.ds(0, ROWS), :],
                    out_sums.at[cid, pl.ds(sid * ROWS, ROWS), :])
    pltpu.sync_copy(hist, out_counts.at[wid])


def _finish_body(s01, cnt, we, be, wo, bo, out):
    sums = s01[0] + s01[1]
    counts = lax.dot_general(cnt[...], jnp.ones((NW, 1), jnp.float32),
                             (((0,), (0,)), ((), ())),
                             precision=lax.Precision.HIGHEST,
                             preferred_element_type=jnp.float32)
    mean = sums * (1.0 / jnp.maximum(counts, 1.0))
    g = lax.dot_general(mean, we[...], (((1,), (0,)), ((), ())),
                        precision=lax.Precision.HIGHEST,
                        preferred_element_type=jnp.float32)
    g = g + jnp.where(counts > 0.0, 1.0, 0.0) * be[...]
    out[...] = lax.dot_general(g, wo[...], (((1,), (0,)), ((), ())),
                               precision=lax.Precision.HIGHEST,
                               preferred_element_type=jnp.float32) + bo[...]


_finish = pl.pallas_call(
    _finish_body,
    out_shape=jax.ShapeDtypeStruct((NUM_SEGMENTS, NUM_PIECES), jnp.float32),
)


def kernel(node_feats, segment_ids, W_embed, b_embed, W_out, b_out):
    ids2 = jnp.pad(segment_ids, (0, IDROWS * TILE - N)).reshape(IDROWS, TILE)
    sums, counts = _segment_sums(node_feats, segment_ids, ids2)
    return _finish(
        sums, counts,
        W_embed, b_embed.reshape(1, D),
        W_out, b_out.reshape(1, NUM_PIECES),
    )
